# baseline (speedup 1.0000x reference)
import functools

import jax
import jax.numpy as jnp
from jax.experimental import pallas as pl
from jax.experimental.pallas import tpu as pltpu
from jax.experimental.pallas import tpu_sc as plsc

_F32 = jnp.float32
_BF16 = jnp.bfloat16
_HI = jax.lax.Precision.HIGHEST


def _argmin_body(zf2_ref, cb2_ref, zbf_ref, cbt_ref, idx_ref, rmin, ridx):
    c = pl.program_id(0)
    nchunk = pl.num_programs(0)
    chunk = cb2_ref.shape[1]
    mm = jax.lax.dot_general(zbf_ref[...], cbt_ref[...],
                             (((1,), (0,)), ((), ())),
                             preferred_element_type=_F32)
    d = (zf2_ref[...] + cb2_ref[...]) - 2.0 * mm
    m = jnp.min(d, axis=1, keepdims=True)
    lane = jax.lax.broadcasted_iota(jnp.int32, d.shape, 1)
    gidx = c * chunk + lane
    am = jnp.min(jnp.where(d == m, gidx, jnp.int32(2 ** 30)),
                 axis=1, keepdims=True)

    @pl.when(c == 0)
    def _():
        rmin[...] = m
        ridx[...] = am

    @pl.when(c > 0)
    def _():
        better = m < rmin[...]
        rmin[...] = jnp.where(better, m, rmin[...])
        ridx[...] = jnp.where(better, am, ridx[...])

    @pl.when(c == nchunk - 1)
    def _():
        idx_ref[...] = ridx[...]


def _argmin_call(zf, codebook):
    n, e = codebook.shape
    b = zf.shape[0]
    chunk = 2048
    zf2 = jnp.sum(zf ** 2, axis=1, keepdims=True)
    cb2 = jnp.sum(codebook ** 2, axis=1)[None, :]
    zpad = jnp.zeros((b, 128 - e), _F32)
    zbf = jnp.concatenate([zf, zpad], axis=1).astype(_BF16)
    cpad = jnp.zeros((n, 128 - e), _F32)
    cbt = jnp.concatenate([codebook, cpad], axis=1).T.astype(_BF16)
    grid = n // chunk
    return pl.pallas_call(
        _argmin_body,
        grid=(grid,),
        in_specs=[
            pl.BlockSpec((b, 1), lambda c: (0, 0)),
            pl.BlockSpec((1, chunk), lambda c: (0, c)),
            pl.BlockSpec((b, 128), lambda c: (0, 0)),
            pl.BlockSpec((128, chunk), lambda c: (0, c)),
        ],
        out_specs=pl.BlockSpec((b, 1), lambda c: (0, 0)),
        out_shape=jax.ShapeDtypeStruct((b, 1), jnp.int32),
        scratch_shapes=[pltpu.VMEM((b, 1), _F32), pltpu.VMEM((b, 1), jnp.int32)],
    )(zf2, cb2, zbf, cbt)


def _sc_gather(table, idx):
    info = plsc.get_sparse_core_info()
    nc, ns = info.num_cores, info.num_subcores
    nw = nc * ns
    v, d = table.shape
    b = idx.shape[0]
    bpw = b // nw
    mesh = plsc.VectorSubcoreMesh(core_axis_name="c", subcore_axis_name="s")

    @functools.partial(
        pl.kernel, mesh=mesh,
        out_type=jax.ShapeDtypeStruct((b, d), _F32),
        scratch_types=[
            pltpu.VMEM((bpw,), jnp.int32),
            pltpu.VMEM((bpw, d), _F32),
            pltpu.SemaphoreType.DMA,
        ],
    )
    def k(table_hbm, idx_hbm, out_hbm, idx_v, rows_v, sem):
        wid = jax.lax.axis_index("s") * nc + jax.lax.axis_index("c")
        base = wid * bpw
        pltpu.sync_copy(idx_hbm.at[pl.ds(base, bpw)], idx_v)
        pltpu.async_copy(table_hbm.at[idx_v], rows_v, sem).wait()
        pltpu.sync_copy(rows_v, out_hbm.at[pl.ds(base, bpw)])

    return k(table, idx)


def _to_padded_flat(x_hwc, H, W):
    return jnp.pad(x_hwc, ((1, 1), (1, 1), (0, 0))).reshape((H + 2) * (W + 2), -1)


def _interior_mask_rows(p, H, W):
    wp = W + 2
    col = p - (p // wp) * wp
    row = p // wp
    return (col >= 1) & (col <= W) & (row >= 1) & (row <= H)


def _gsc_body(x_ref, gb_ref, w_ref, b_ref, res_ref, out_ref, act, win, ss,
              *, H, W, Cin, Cout, bs, norm, swish, has_res, groups):
    wp = W + 2
    npad = (H + 2) * wp
    mg = -(-(wp + 1) // 16) * 16
    i = pl.program_id(0)

    ck = 2048
    nchunk = -(-npad // ck)

    @pl.when(i == 0)
    def _():
        if norm:
            gs = Cin // groups
            ch = jax.lax.broadcasted_iota(jnp.int32, (Cin, groups), 0)
            gr = jax.lax.broadcasted_iota(jnp.int32, (Cin, groups), 1)
            mmat = (ch // gs == gr).astype(_F32)
            s1 = jnp.zeros((1, Cin), _F32)
            s2 = jnp.zeros((1, Cin), _F32)
            for c in range(nchunk):
                ln = min(ck, npad - c * ck)
                xc = x_ref[c * ck:c * ck + ln, :]
                s1 = s1 + jnp.sum(xc, axis=0, keepdims=True)
                s2 = s2 + jnp.sum(xc * xc, axis=0, keepdims=True)
            cnt = float(H * W * gs)
            g1 = jax.lax.dot_general(s1, mmat, (((1,), (0,)), ((), ())),
                                     precision=_HI, preferred_element_type=_F32) / cnt
            g2 = jax.lax.dot_general(s2, mmat, (((1,), (0,)), ((), ())),
                                     precision=_HI, preferred_element_type=_F32) / cnt
            var = g2 - g1 * g1
            rstd = 1.0 / jnp.sqrt(var + 1e-6)
            mean_c = jax.lax.dot_general(g1, mmat, (((1,), (1,)), ((), ())),
                                         precision=_HI, preferred_element_type=_F32)
            rstd_c = jax.lax.dot_general(rstd, mmat, (((1,), (1,)), ((), ())),
                                         precision=_HI, preferred_element_type=_F32)
            scale = gb_ref[0:1, :] * rstd_c
            shift = gb_ref[1:2, :] - mean_c * scale
        else:
            scale = jnp.ones((1, Cin), _F32)
            shift = jnp.zeros((1, Cin), _F32)
        ss[0:1, :] = scale
        ss[1:2, :] = shift
        act[...] = jnp.zeros(act.shape, _BF16)
        for c in range(nchunk):
            ln = min(ck, npad - c * ck)
            u = x_ref[c * ck:c * ck + ln, :]
            if norm:
                u = u * scale + shift
            if swish:
                u = u * (1.0 / (1.0 + jnp.exp(-u)))
            p = c * ck + jax.lax.broadcasted_iota(jnp.int32, (ln, 1), 0)
            keep = _interior_mask_rows(p, H, W)
            u = jnp.where(keep, u, 0.0)
            act[mg + c * ck:mg + c * ck + ln, :] = u.astype(_BF16)

    r0 = pl.multiple_of(i * bs, 16)
    win[...] = act[pl.ds(r0, bs + 2 * mg), :]
    acc = jnp.zeros((bs, Cout), _F32)
    for t in range(9):
        dh, dw = t // 3 - 1, t % 3 - 1
        a = win[mg + dh * wp + dw:mg + dh * wp + dw + bs, :]
        acc = acc + jax.lax.dot_general(
            a, w_ref[t], (((1,), (0,)), ((), ())),
            preferred_element_type=_F32)
    p = i * bs + jax.lax.broadcasted_iota(jnp.int32, (bs, 1), 0)
    keep = _interior_mask_rows(p, H, W)
    y = jnp.where(keep, acc + b_ref[...], 0.0)
    if has_res:
        y = y + res_ref[...]
    out_ref[...] = y


def _gsc_call(x, gb, w9, bias, res, *, H, W, Cin, Cout,
              norm=True, swish=True, groups=32):
    wp = W + 2
    npad = (H + 2) * wp
    mg = -(-(wp + 1) // 16) * 16
    bs = min(npad, 2048)
    nblocks = -(-npad // bs)
    sp = nblocks * bs + 2 * mg
    has_res = res is not None
    if gb is None:
        gb = jnp.zeros((2, Cin), _F32)
    body = functools.partial(_gsc_body, H=H, W=W, Cin=Cin, Cout=Cout, bs=bs,
                             norm=norm, swish=swish, has_res=has_res,
                             groups=groups)
    in_specs = [
        pl.BlockSpec((npad, Cin), lambda i: (0, 0)),
        pl.BlockSpec((2, Cin), lambda i: (0, 0)),
        pl.BlockSpec((9, Cin, Cout), lambda i: (0, 0, 0)),
        pl.BlockSpec((1, Cout), lambda i: (0, 0)),
    ]
    args = [x, gb, w9, bias]
    if has_res:
        in_specs.append(pl.BlockSpec((bs, Cout), lambda i: (i, 0)))
        args.append(res)
    else:
        in_specs.append(pl.BlockSpec((1, Cout), lambda i: (0, 0)))
        args.append(bias)

    def body_wrap(x_ref, gb_ref, w_ref, b_ref, res_ref, out_ref, act, win, ss):
        body(x_ref, gb_ref, w_ref, b_ref, res_ref, out_ref, act, win, ss)

    return pl.pallas_call(
        body_wrap,
        grid=(nblocks,),
        in_specs=in_specs,
        out_specs=pl.BlockSpec((bs, Cout), lambda i: (i, 0)),
        out_shape=jax.ShapeDtypeStruct((npad, Cout), _F32),
        scratch_shapes=[pltpu.VMEM((sp, Cin), _BF16),
                        pltpu.VMEM((bs + 2 * mg, Cin), _BF16),
                        pltpu.VMEM((2, Cin), _F32)],
    )(*args)


def _c1_body(x_ref, w_ref, b_ref, out_ref, *, H, W):
    npad = x_ref.shape[0]
    xb = x_ref[...].astype(_BF16)
    y = jax.lax.dot_general(xb, w_ref[...], (((1,), (0,)), ((), ())),
                            preferred_element_type=_F32)
    p = jax.lax.broadcasted_iota(jnp.int32, (npad, 1), 0)
    keep = _interior_mask_rows(p, H, W)
    out_ref[...] = jnp.where(keep, y + b_ref[...], 0.0)


def _c1_call(x, w, bias, *, H, W):
    npad, cin = x.shape
    cout = w.shape[1]
    return pl.pallas_call(
        functools.partial(_c1_body, H=H, W=W),
        out_shape=jax.ShapeDtypeStruct((npad, cout), _F32),
    )(x, w, bias)


def _attn_body(x_ref, gb_ref, wq, wk, wv, wp_, bq, bk, bv, bp_, out_ref,
               *, H, W, C, groups):
    npad = x_ref.shape[0]
    x = x_ref[...]
    gs = C // groups
    ch = jax.lax.broadcasted_iota(jnp.int32, (C, groups), 0)
    gr = jax.lax.broadcasted_iota(jnp.int32, (C, groups), 1)
    mmat = (ch // gs == gr).astype(_F32)
    s1 = jnp.sum(x, axis=0, keepdims=True)
    s2 = jnp.sum(x * x, axis=0, keepdims=True)
    cnt = float(H * W * gs)
    g1 = jax.lax.dot_general(s1, mmat, (((1,), (0,)), ((), ())),
                             precision=_HI, preferred_element_type=_F32) / cnt
    g2 = jax.lax.dot_general(s2, mmat, (((1,), (0,)), ((), ())),
                             precision=_HI, preferred_element_type=_F32) / cnt
    var = g2 - g1 * g1
    rstd = 1.0 / jnp.sqrt(var + 1e-6)
    mean_c = jax.lax.dot_general(g1, mmat, (((1,), (1,)), ((), ())),
                                 precision=_HI, preferred_element_type=_F32)
    rstd_c = jax.lax.dot_general(rstd, mmat, (((1,), (1,)), ((), ())),
                                 precision=_HI, preferred_element_type=_F32)
    scale = gb_ref[0:1, :] * rstd_c
    shift = gb_ref[1:2, :] - mean_c * scale
    p = jax.lax.broadcasted_iota(jnp.int32, (npad, 1), 0)
    keep = _interior_mask_rows(p, H, W)
    hn = jnp.where(keep, x * scale + shift, 0.0).astype(_BF16)

    q = jax.lax.dot_general(hn, wq[...], (((1,), (0,)), ((), ())),
                            preferred_element_type=_F32) + bq[...]
    k = jax.lax.dot_general(hn, wk[...], (((1,), (0,)), ((), ())),
                            preferred_element_type=_F32) + bk[...]
    v = jax.lax.dot_general(hn, wv[...], (((1,), (0,)), ((), ())),
                            preferred_element_type=_F32) + bv[...]
    s = jax.lax.dot_general(q.astype(_BF16), k.astype(_BF16),
                            (((1,), (1,)), ((), ())),
                            preferred_element_type=_F32) * (float(C) ** -0.5)
    kmask = _interior_mask_rows(
        jax.lax.broadcasted_iota(jnp.int32, (1, npad), 1), H, W)
    s = jnp.where(kmask, s, -1e30)
    m = jnp.max(s, axis=1, keepdims=True)
    e = jnp.exp(s - m)
    w_ = e / jnp.sum(e, axis=1, keepdims=True)
    hv = jax.lax.dot_general(w_.astype(_BF16), v.astype(_BF16),
                             (((1,), (0,)), ((), ())),
                             preferred_element_type=_F32)
    o = jax.lax.dot_general(hv.astype(_BF16), wp_[...], (((1,), (0,)), ((), ())),
                            preferred_element_type=_F32) + bp_[...]
    out_ref[...] = jnp.where(keep, x + o, 0.0)


def _attn_call(x, gb, wq, wk, wv, wp_, bq, bk, bv, bp_, *, H, W, C):
    npad = x.shape[0]
    return pl.pallas_call(
        functools.partial(_attn_body, H=H, W=W, C=C, groups=32),
        out_shape=jax.ShapeDtypeStruct((npad, C), _F32),
    )(x, gb, wq, wk, wv, wp_, bq, bk, bv, bp_)


def _prep3(p):
    w = p["w"]
    cout, cin = w.shape[0], w.shape[1]
    w9 = jnp.transpose(w.reshape(cout, cin, 9), (2, 1, 0)).astype(_BF16)
    return w9, p["b"][None, :]


def _prep1(p):
    w = p["w"]
    return jnp.transpose(w[:, :, 0, 0], (1, 0)).astype(_BF16), p["b"][None, :]


def _gb(p):
    return jnp.stack([p["g"], p["b"]])


def _resblock(h, p, H, W, cin, cout):
    w1, b1 = _prep3(p["conv1"])
    w2, b2 = _prep3(p["conv2"])
    h1 = _gsc_call(h, _gb(p["norm1"]), w1, b1, None,
                   H=H, W=W, Cin=cin, Cout=cout)
    if "nin" in p:
        wn, bn = _prep1(p["nin"])
        res = _c1_call(h, wn, bn, H=H, W=W)
    else:
        res = h
    return _gsc_call(h1, _gb(p["norm2"]), w2, b2, res,
                     H=H, W=W, Cin=cout, Cout=cout)


def _upsample(h, H, W, C):
    hi = h.reshape(H + 2, W + 2, C)[1:H + 1, 1:W + 1, :]
    hi = jnp.repeat(jnp.repeat(hi, 2, axis=0), 2, axis=1)
    return _to_padded_flat(hi, 2 * H, 2 * W)


def kernel(x, codebook, params):
    zp = jnp.transpose(x, (0, 2, 3, 1))
    zf = zp.reshape(-1, 3)
    idx = _argmin_call(zf, codebook)[:, 0]
    cb_pad = jnp.pad(codebook, ((0, 0), (0, 125)))
    zq = _sc_gather(cb_pad, idx)[:, :3]
    quant = zp.reshape(-1, 3) + (zq - zp.reshape(-1, 3))
    h = _to_padded_flat(quant.reshape(32, 32, 3), 32, 32)

    wpq, bpq = _prep1(params["post_quant_conv"])
    h = _c1_call(h, wpq, bpq, H=32, W=32)
    wci, bci = _prep3(params["conv_in"])
    h = _gsc_call(h, None, wci, bci, None, H=32, W=32, Cin=3, Cout=512,
                  norm=False, swish=False)

    mid = params["mid"]
    h = _resblock(h, mid["block_1"], 32, 32, 512, 512)
    ap = mid["attn_1"]
    wq, bq = _prep1(ap["q"])
    wk, bk = _prep1(ap["k"])
    wv, bv = _prep1(ap["v"])
    wo, bo = _prep1(ap["proj_out"])
    h = _attn_call(h, _gb(ap["norm"]), wq, wk, wv, wo, bq, bk, bv, bo,
                   H=32, W=32, C=512)
    h = _resblock(h, mid["block_2"], 32, 32, 512, 512)

    ch_mult = (1, 2, 4)
    ch = 128
    res_hw = 32
    block_in = ch * ch_mult[-1]
    for i_level in reversed(range(len(ch_mult))):
        lvl = params["up"][i_level]
        block_out = ch * ch_mult[i_level]
        for bp in lvl["blocks"]:
            h = _resblock(h, bp, res_hw, res_hw, block_in, block_out)
            block_in = block_out
        if i_level != 0:
            h = _upsample(h, res_hw, res_hw, block_in)
            res_hw *= 2
            wu, bu = _prep3(lvl["upsample"])
            h = _gsc_call(h, None, wu, bu, None, H=res_hw, W=res_hw,
                          Cin=block_in, Cout=block_in, norm=False, swish=False)

    wo9, bo9 = _prep3(params["conv_out"])
    h = _gsc_call(h, _gb(params["norm_out"]), wo9, bo9, None,
                  H=res_hw, W=res_hw, Cin=block_in, Cout=3)
    out = h.reshape(res_hw + 2, res_hw + 2, 3)[1:res_hw + 1, 1:res_hw + 1, :]
    return jnp.transpose(out, (2, 0, 1))[None]
